# baseline (speedup 1.0000x reference)
import functools
import math

import jax
import jax.numpy as jnp
from jax import lax
from jax.experimental import pallas as pl
from jax.experimental.pallas import tpu as pltpu
from jax.experimental.pallas import tpu_sc as plsc

DIM = 64
_SCALE = math.sqrt(DIM)

NC = 2
NS = 16
NW = NC * NS

GROUP = 128
NROW = 4
LOOKAHEAD = 2
TP = 128


@jax.jit
def _embed_lookup(table, idx2d):
    ngroups = idx2d.shape[0]
    g_per_w = ngroups // NW
    sdim = 200
    btiles = 32
    dtiles = DIM // 8
    mesh = plsc.VectorSubcoreMesh(core_axis_name="c", subcore_axis_name="s")

    @functools.partial(
        pl.kernel,
        mesh=mesh,
        out_type=jax.ShapeDtypeStruct((sdim, dtiles, btiles, 8, GROUP),
                                      jnp.float32),
        scratch_types=[
            pltpu.VMEM((g_per_w, GROUP), jnp.int32),
            pltpu.VMEM((NROW, GROUP, DIM), jnp.float32),
            pltpu.VMEM((2, DIM, TP), jnp.float32),
            pltpu.SemaphoreType.DMA((NROW,)),
            pltpu.SemaphoreType.DMA((2,)),
        ],
        compiler_params=pltpu.CompilerParams(use_tc_tiling_on_sc=False,
                                             needs_layout_passes=False),
    )
    def k(table_hbm, idx_hbm, out_hbm, idx_all, rows_v, trans_v, gsem, osem):
        wid = lax.axis_index("s") * NC + lax.axis_index("c")
        r0 = wid * g_per_w

        pltpu.sync_copy(idx_hbm.at[pl.ds(r0, g_per_w)], idx_all)

        for q in range(LOOKAHEAD):
            pltpu.async_copy(table_hbm.at[idx_all.at[q]], rows_v.at[q],
                             gsem.at[q])

        lane = jnp.arange(16, dtype=jnp.int32)

        @pl.loop(0, g_per_w // NROW)
        def outer(p):
            for q in range(NROW):
                g = p * NROW + q
                tb = q % 2

                pltpu.make_async_copy(
                    table_hbm.at[pl.ds(0, GROUP)], rows_v.at[q],
                    gsem.at[q]).wait()

                nq = (q + LOOKAHEAD) % NROW
                gn = g + LOOKAHEAD

                @pl.when(gn < g_per_w)
                def _prefetch():
                    pltpu.async_copy(
                        table_hbm.at[idx_all.at[gn]], rows_v.at[nq],
                        gsem.at[nq])

                r = r0 + g
                ss = lax.rem(r, 8)
                bt = lax.rem(lax.div(r, 8), btiles)
                s = lax.div(r, 8 * btiles) * 8 + ss

                @pl.when(g >= 2)
                def _drain():
                    pltpu.make_async_copy(
                        trans_v.at[tb, :, pl.ds(0, GROUP)],
                        out_hbm.at[0, :, 0], osem.at[tb]).wait()

                @plsc.parallel_loop(0, GROUP, unroll=4)
                def transpose(t):
                    col = jnp.full((16,), t, dtype=jnp.int32)
                    for c in range(DIM // 16):
                        v = rows_v[q, t, pl.ds(c * 16, 16)] * _SCALE
                        plsc.store_scatter(
                            trans_v.at[tb], [c * 16 + lane, col], v)

                for dt in range(dtiles):
                    pltpu.async_copy(
                        trans_v.at[tb, pl.ds(dt * 8, 8), pl.ds(0, GROUP)],
                        out_hbm.at[s, dt, bt], osem.at[tb])

        for tb in range(2):
            pltpu.make_async_copy(
                trans_v.at[tb, :, pl.ds(0, GROUP)],
                out_hbm.at[0, :, 0], osem.at[tb]).wait()

    return k(table, idx2d)


def kernel(input_ids, embedding):
    nb, ns = input_ids.shape
    idx2d = (input_ids.T.reshape(ns // 8, 8, nb // 128, 128)
             .transpose(0, 2, 1, 3)
             .reshape(ns // 8 * (nb // 128) * 8, 128)
             .astype(jnp.int32))
    out5d = _embed_lookup(embedding, idx2d)
    return out5d.transpose(2, 4, 0, 1, 3).reshape(nb, ns, DIM)


# speedup vs baseline: 1.3510x; 1.3510x over previous
import functools
import math

import jax
import jax.numpy as jnp
from jax import lax
from jax.experimental import pallas as pl
from jax.experimental.pallas import tpu as pltpu
from jax.experimental.pallas import tpu_sc as plsc

DIM = 64
_SCALE = math.sqrt(DIM)

NC = 2
NS = 16
NW = NC * NS

GROUP = 128
NROW = 4
LOOKAHEAD = 2
TP = 129


def _transpose_table(table_t, tail128):
    v = table_t.shape[1]
    nfull = v // 128
    mesh = plsc.VectorSubcoreMesh(core_axis_name="c", subcore_axis_name="s")

    @functools.partial(
        pl.kernel,
        mesh=mesh,
        out_type=jax.ShapeDtypeStruct((v // 2, 128), jnp.float32),
        scratch_types=[
            pltpu.VMEM((2, 8, 8, TP), jnp.float32),
            pltpu.VMEM((2, 64, 128), jnp.float32),
            pltpu.VMEM((64, TP), jnp.float32),
            pltpu.SemaphoreType.DMA((2,)),
            pltpu.SemaphoreType.DMA((2,)),
        ],
        compiler_params=pltpu.CompilerParams(use_tc_tiling_on_sc=True,
                                             needs_layout_passes=False),
    )
    def k(tab_hbm, tail_hbm, out_hbm, slab_v, pack_v, tail_v, gsem, osem):
        wid = lax.axis_index("s") * NC + lax.axis_index("c")
        lane = jnp.arange(16, dtype=jnp.int32)
        g_hi = lane // 8
        d_lo = lane % 8

        def fire(blk, k_):
            for g in range(8):
                pltpu.async_copy(
                    tab_hbm.at[pl.ds(g * 8, 8), pl.ds(blk * 128, 128)],
                    slab_v.at[k_, g, :, pl.ds(0, 128)], gsem.at[k_])

        fire(wid, 0)

        niter = nfull // NW + 2

        @pl.loop(0, niter)
        def outer(i):
            k_ = lax.rem(i, 2)
            blk = i * NW + wid
            nblk_ = blk + NW

            for kk in range(2):

                @pl.when(jnp.logical_and(k_ == kk, nblk_ < nfull))
                def _pref():
                    fire(nblk_, 1 - kk)

                @pl.when(jnp.logical_and(k_ == kk, blk < nfull))
                def _work():
                    pltpu.make_async_copy(
                        tab_hbm.at[pl.ds(0, 8), pl.ds(0, 128)],
                        slab_v.at[kk, :, :, pl.ds(0, 128)],
                        gsem.at[kk]).wait()

                    @pl.when(i >= 2)
                    def _drain():
                        pltpu.make_async_copy(
                            pack_v.at[kk], out_hbm.at[pl.ds(0, 64), :],
                            osem.at[kk]).wait()

                    @plsc.parallel_loop(0, 128, unroll=4)
                    def body(vv):
                        col = jnp.full((16,), vv, dtype=jnp.int32)
                        half = lax.rem(vv, 2) * 64
                        row = lax.div(vv, 2)
                        for c in range(4):
                            vals = plsc.load_gather(
                                slab_v.at[kk],
                                [g_hi + 2 * c, d_lo, col])
                            pack_v[kk, row, pl.ds(half + c * 16, 16)] = vals

                    pltpu.async_copy(pack_v.at[kk],
                                     out_hbm.at[pl.ds(blk * 64, 64), :],
                                     osem.at[kk])

        for kk in range(2):
            pltpu.make_async_copy(
                pack_v.at[kk], out_hbm.at[pl.ds(0, 64), :],
                osem.at[kk]).wait()

        @pl.when(wid == 0)
        def _tail():
            pltpu.sync_copy(tail_hbm, tail_v.at[:, pl.ds(0, 128)])

            @plsc.parallel_loop(64, 128, unroll=4)
            def tail_body(vv):
                col = jnp.full((16,), vv, dtype=jnp.int32)
                half = lax.rem(vv, 2) * 64
                row = lax.div(vv, 2) - 32
                for c in range(4):
                    vals = plsc.load_gather(tail_v, [c * 16 + lane, col])
                    pack_v[0, row, pl.ds(half + c * 16, 16)] = vals

            pltpu.sync_copy(pack_v.at[0, pl.ds(0, 32), :],
                            out_hbm.at[pl.ds(nfull * 64, 32), :])

    return k(table_t, tail128)


def _gather_scale(table, idx2d):
    ngroups = idx2d.shape[0]
    g_per_w = ngroups // NW
    sdim = 200
    btiles = 32
    dtiles = DIM // 8
    mesh = plsc.VectorSubcoreMesh(core_axis_name="c", subcore_axis_name="s")

    @functools.partial(
        pl.kernel,
        mesh=mesh,
        out_type=jax.ShapeDtypeStruct((sdim, dtiles, btiles, 8, GROUP),
                                      jnp.float32),
        scratch_types=[
            pltpu.VMEM((g_per_w, GROUP), jnp.int32),
            pltpu.VMEM((NROW, GROUP, DIM), jnp.float32),
            pltpu.VMEM((2, DIM, TP), jnp.float32),
            pltpu.SemaphoreType.DMA((NROW,)),
            pltpu.SemaphoreType.DMA((2,)),
        ],
        compiler_params=pltpu.CompilerParams(use_tc_tiling_on_sc=False,
                                             needs_layout_passes=False),
    )
    def k(table_hbm, idx_hbm, out_hbm, idx_all, rows_v, trans_v, gsem, osem):
        wid = lax.axis_index("s") * NC + lax.axis_index("c")
        r0 = wid * g_per_w

        pltpu.sync_copy(idx_hbm.at[pl.ds(r0, g_per_w)], idx_all)

        for q in range(LOOKAHEAD):
            pltpu.async_copy(table_hbm.at[idx_all.at[q]], rows_v.at[q],
                             gsem.at[q])

        lane = jnp.arange(16, dtype=jnp.int32)

        @pl.loop(0, g_per_w // NROW)
        def outer(p):
            for q in range(NROW):
                g = p * NROW + q
                tb = q % 2

                pltpu.make_async_copy(
                    table_hbm.at[pl.ds(0, GROUP)], rows_v.at[q],
                    gsem.at[q]).wait()

                nq = (q + LOOKAHEAD) % NROW
                gn = g + LOOKAHEAD

                @pl.when(gn < g_per_w)
                def _prefetch():
                    pltpu.async_copy(
                        table_hbm.at[idx_all.at[gn]], rows_v.at[nq],
                        gsem.at[nq])

                r = r0 + g
                ss = lax.rem(r, 8)
                bt = lax.rem(lax.div(r, 8), btiles)
                s = lax.div(r, 8 * btiles) * 8 + ss

                @pl.when(g >= 2)
                def _drain():
                    pltpu.make_async_copy(
                        trans_v.at[tb, :, pl.ds(0, GROUP)],
                        out_hbm.at[0, :, 0], osem.at[tb]).wait()

                @plsc.parallel_loop(0, GROUP, unroll=4)
                def transpose(t):
                    col = jnp.full((16,), t, dtype=jnp.int32)
                    for c in range(DIM // 16):
                        v = rows_v[q, t, pl.ds(c * 16, 16)] * _SCALE
                        plsc.store_scatter(
                            trans_v.at[tb], [c * 16 + lane, col], v)

                for dt in range(dtiles):
                    pltpu.async_copy(
                        trans_v.at[tb, pl.ds(dt * 8, 8), pl.ds(0, GROUP)],
                        out_hbm.at[s, dt, bt], osem.at[tb])

        for tb in range(2):
            pltpu.make_async_copy(
                trans_v.at[tb, :, pl.ds(0, GROUP)],
                out_hbm.at[0, :, 0], osem.at[tb]).wait()

    return k(table, idx2d)


@jax.jit
def _embed_lookup(table_t, tail128, idx2d):
    packed = _transpose_table(table_t, tail128)
    table_dense = packed.reshape(table_t.shape[1], DIM)
    return _gather_scale(table_dense, idx2d)


def kernel(input_ids, embedding):
    nb, ns = input_ids.shape
    nv = embedding.shape[0]
    idx2d = (input_ids.T.reshape(ns // 8, 8, nb // 128, 128)
             .transpose(0, 2, 1, 3)
             .reshape(ns // 8 * (nb // 128) * 8, 128)
             .astype(jnp.int32))
    tail128 = embedding[nv - 128:, :].T
    out5d = _embed_lookup(embedding.T, tail128, idx2d)
    return out5d.transpose(2, 4, 0, 1, 3).reshape(nb, ns, DIM)


# speedup vs baseline: 2.0983x; 1.5531x over previous
import functools
import math

import jax
import jax.numpy as jnp
from jax import lax
from jax.experimental import pallas as pl
from jax.experimental.pallas import tpu as pltpu
from jax.experimental.pallas import tpu_sc as plsc

DIM = 64
_SCALE = math.sqrt(DIM)

NC = 2
NS = 16
NW = NC * NS

GROUP = 128
NROW = 4
LOOKAHEAD = 2
TP = 129
KT = 16


def _restack_tiles(table_t):
    v = table_t.shape[1]
    nfull = v // 128
    nchunk = nfull // KT
    rem = nfull - nchunk * KT
    mesh = plsc.VectorSubcoreMesh(core_axis_name="c", subcore_axis_name="s")

    @functools.partial(
        pl.kernel,
        mesh=mesh,
        out_type=jax.ShapeDtypeStruct((8, nfull, 8, 128), jnp.float32),
        scratch_types=[
            pltpu.VMEM((2, 8, 128 * KT), jnp.float32),
            pltpu.SemaphoreType.DMA((2,)),
            pltpu.SemaphoreType.DMA((2,)),
        ],
        compiler_params=pltpu.CompilerParams(use_tc_tiling_on_sc=True,
                                             needs_layout_passes=False),
    )
    def k(tab_hbm, out_hbm, buf_v, gsem, osem):
        wid = lax.axis_index("s") * NC + lax.axis_index("c")
        nunits = 8 * nchunk

        def fire(u, k_):
            g = lax.rem(u, 8)
            i = lax.div(u, 8)
            pltpu.async_copy(
                tab_hbm.at[pl.ds(g * 8, 8), pl.ds(i * (128 * KT), 128 * KT)],
                buf_v.at[k_], gsem.at[k_])

        fire(wid, 0)

        niters = nunits // NW

        @pl.loop(0, niters)
        def outer(it):
            u = it * NW + wid
            for kk in range(2):

                @pl.when(lax.rem(it, 2) == kk)
                def _work():
                    pltpu.make_async_copy(
                        tab_hbm.at[pl.ds(0, 8), pl.ds(0, 128 * KT)],
                        buf_v.at[kk], gsem.at[kk]).wait()

                    @pl.when(it >= 1)
                    def _drain():
                        for j in range(KT):
                            pltpu.make_async_copy(
                                buf_v.at[1 - kk, :, pl.ds(0, 128)],
                                out_hbm.at[0, 0], osem.at[1 - kk]).wait()

                    @pl.when(it + 1 < niters)
                    def _pref():
                        fire(u + NW, 1 - kk)

                    g = lax.rem(u, 8)
                    i = lax.div(u, 8)
                    for j in range(KT):
                        pltpu.async_copy(
                            buf_v.at[kk, :, pl.ds(j * 128, 128)],
                            out_hbm.at[g, i * KT + j], osem.at[kk])

        for j in range(KT):
            pltpu.make_async_copy(
                buf_v.at[(niters - 1) % 2, :, pl.ds(0, 128)],
                out_hbm.at[0, 0], osem.at[(niters - 1) % 2]).wait()

        @pl.when(wid < 8 * rem)
        def _left():
            g = lax.rem(wid, 8)
            c = nchunk * KT + lax.div(wid, 8)
            pltpu.sync_copy(
                tab_hbm.at[pl.ds(g * 8, 8), pl.ds(c * 128, 128)],
                buf_v.at[0, :, pl.ds(0, 128)])
            pltpu.sync_copy(buf_v.at[0, :, pl.ds(0, 128)], out_hbm.at[g, c])

    return k(table_t)


def _transpose_blocks(x2, tail128):
    nfull = x2.shape[1]
    v = (nfull + 1) * 128
    mesh = plsc.VectorSubcoreMesh(core_axis_name="c", subcore_axis_name="s")

    @functools.partial(
        pl.kernel,
        mesh=mesh,
        out_type=jax.ShapeDtypeStruct((1000000 // 2, 128), jnp.float32),
        scratch_types=[
            pltpu.VMEM((2, 8, 8, TP), jnp.float32),
            pltpu.VMEM((2, 64, 128), jnp.float32),
            pltpu.VMEM((64, TP), jnp.float32),
            pltpu.SemaphoreType.DMA((2,)),
            pltpu.SemaphoreType.DMA((2,)),
        ],
        compiler_params=pltpu.CompilerParams(use_tc_tiling_on_sc=False,
                                             needs_layout_passes=False),
    )
    def k(x2_hbm, tail_hbm, out_hbm, slab_v, pack_v, tail_v, gsem, osem):
        wid = lax.axis_index("s") * NC + lax.axis_index("c")
        lane = jnp.arange(16, dtype=jnp.int32)
        g_hi = lane // 8
        d_lo = lane % 8

        def fire(blk, k_):
            pltpu.async_copy(
                x2_hbm.at[:, blk, :, :],
                slab_v.at[k_, :, :, pl.ds(0, 128)], gsem.at[k_])

        fire(wid, 0)

        niter = nfull // NW + 2

        @pl.loop(0, niter)
        def outer(i):
            blk = i * NW + wid
            for kk in range(2):

                @pl.when(jnp.logical_and(lax.rem(i, 2) == kk, blk < nfull))
                def _work():
                    nblk = blk + NW

                    @pl.when(nblk < nfull)
                    def _pref():
                        fire(nblk, 1 - kk)

                    pltpu.make_async_copy(
                        x2_hbm.at[:, 0, :, :],
                        slab_v.at[kk, :, :, pl.ds(0, 128)],
                        gsem.at[kk]).wait()

                    @pl.when(i >= 2)
                    def _drain():
                        pltpu.make_async_copy(
                            pack_v.at[kk], out_hbm.at[pl.ds(0, 64), :],
                            osem.at[kk]).wait()

                    @plsc.parallel_loop(0, 128, unroll=4)
                    def body(vv):
                        col = jnp.full((16,), vv, dtype=jnp.int32)
                        half = lax.rem(vv, 2) * 64
                        row = lax.div(vv, 2)
                        for c in range(4):
                            vals = plsc.load_gather(
                                slab_v.at[kk], [g_hi + 2 * c, d_lo, col])
                            pack_v[kk, row, pl.ds(half + c * 16, 16)] = vals

                    pltpu.async_copy(pack_v.at[kk],
                                     out_hbm.at[pl.ds(blk * 64, 64), :],
                                     osem.at[kk])

        for kk in range(2):
            pltpu.make_async_copy(
                pack_v.at[kk], out_hbm.at[pl.ds(0, 64), :],
                osem.at[kk]).wait()

        @pl.when(wid == 0)
        def _tail():
            pltpu.sync_copy(tail_hbm, tail_v.at[:, pl.ds(0, 128)])

            @plsc.parallel_loop(64, 128, unroll=4)
            def tail_body(vv):
                col = jnp.full((16,), vv, dtype=jnp.int32)
                half = lax.rem(vv, 2) * 64
                row = lax.div(vv, 2) - 32
                for c in range(4):
                    vals = plsc.load_gather(tail_v, [c * 16 + lane, col])
                    pack_v[0, row, pl.ds(half + c * 16, 16)] = vals

            pltpu.sync_copy(pack_v.at[0, pl.ds(0, 32), :],
                            out_hbm.at[pl.ds(nfull * 64, 32), :])

    return k(x2, tail128)


def _gather_scale(table, idx2d):
    ngroups = idx2d.shape[0]
    g_per_w = ngroups // NW
    sdim = 200
    btiles = 32
    dtiles = DIM // 8
    mesh = plsc.VectorSubcoreMesh(core_axis_name="c", subcore_axis_name="s")

    @functools.partial(
        pl.kernel,
        mesh=mesh,
        out_type=jax.ShapeDtypeStruct((sdim, dtiles, btiles, 8, GROUP),
                                      jnp.float32),
        scratch_types=[
            pltpu.VMEM((g_per_w, GROUP), jnp.int32),
            pltpu.VMEM((NROW, GROUP, DIM), jnp.float32),
            pltpu.VMEM((2, DIM, TP), jnp.float32),
            pltpu.SemaphoreType.DMA((NROW,)),
            pltpu.SemaphoreType.DMA((2,)),
        ],
        compiler_params=pltpu.CompilerParams(use_tc_tiling_on_sc=False,
                                             needs_layout_passes=False),
    )
    def k(table_hbm, idx_hbm, out_hbm, idx_all, rows_v, trans_v, gsem, osem):
        wid = lax.axis_index("s") * NC + lax.axis_index("c")
        r0 = wid * g_per_w

        pltpu.sync_copy(idx_hbm.at[pl.ds(r0, g_per_w)], idx_all)

        for q in range(LOOKAHEAD):
            pltpu.async_copy(table_hbm.at[idx_all.at[q]], rows_v.at[q],
                             gsem.at[q])

        lane = jnp.arange(16, dtype=jnp.int32)

        @pl.loop(0, g_per_w // NROW)
        def outer(p):
            for q in range(NROW):
                g = p * NROW + q
                tb = q % 2

                pltpu.make_async_copy(
                    table_hbm.at[pl.ds(0, GROUP)], rows_v.at[q],
                    gsem.at[q]).wait()

                nq = (q + LOOKAHEAD) % NROW
                gn = g + LOOKAHEAD

                @pl.when(gn < g_per_w)
                def _prefetch():
                    pltpu.async_copy(
                        table_hbm.at[idx_all.at[gn]], rows_v.at[nq],
                        gsem.at[nq])

                r = r0 + g
                ss = lax.rem(r, 8)
                bt = lax.rem(lax.div(r, 8), btiles)
                s = lax.div(r, 8 * btiles) * 8 + ss

                @pl.when(g >= 2)
                def _drain():
                    pltpu.make_async_copy(
                        trans_v.at[tb, :, pl.ds(0, GROUP)],
                        out_hbm.at[0, :, 0], osem.at[tb]).wait()

                @plsc.parallel_loop(0, GROUP, unroll=4)
                def transpose(t):
                    col = jnp.full((16,), t, dtype=jnp.int32)
                    for c in range(DIM // 16):
                        v = rows_v[q, t, pl.ds(c * 16, 16)] * _SCALE
                        plsc.store_scatter(
                            trans_v.at[tb], [c * 16 + lane, col], v)

                for dt in range(dtiles):
                    pltpu.async_copy(
                        trans_v.at[tb, pl.ds(dt * 8, 8), pl.ds(0, GROUP)],
                        out_hbm.at[s, dt, bt], osem.at[tb])

        for tb in range(2):
            pltpu.make_async_copy(
                trans_v.at[tb, :, pl.ds(0, GROUP)],
                out_hbm.at[0, :, 0], osem.at[tb]).wait()

    return k(table, idx2d)


@jax.jit
def _embed_lookup(table_t, tail128, idx2d):
    x2 = _restack_tiles(table_t)
    packed = _transpose_blocks(x2, tail128)
    table_dense = packed.reshape(table_t.shape[1], DIM)
    return _gather_scale(table_dense, idx2d)


def kernel(input_ids, embedding):
    nb, ns = input_ids.shape
    nv = embedding.shape[0]
    idx2d = (input_ids.T.reshape(ns // 8, 8, nb // 128, 128)
             .transpose(0, 2, 1, 3)
             .reshape(ns // 8 * (nb // 128) * 8, 128)
             .astype(jnp.int32))
    tail128 = embedding[nv - 128:, :].T
    out5d = _embed_lookup(embedding.T, tail128, idx2d)
    return out5d.transpose(2, 4, 0, 1, 3).reshape(nb, ns, DIM)


# speedup vs baseline: 2.1409x; 1.0203x over previous
import functools
import math

import jax
import jax.numpy as jnp
from jax import lax
from jax.experimental import pallas as pl
from jax.experimental.pallas import tpu as pltpu
from jax.experimental.pallas import tpu_sc as plsc

DIM = 64
_SCALE = math.sqrt(DIM)

NC = 2
NS = 16
NW = NC * NS

GROUP = 128
NROW = 4
LOOKAHEAD = 2
TP = 129
KT = 16


def _restack_tiles(table_t):
    v = table_t.shape[1]
    nfull = v // 128
    nchunk = nfull // KT
    rem = nfull - nchunk * KT
    mesh = plsc.VectorSubcoreMesh(core_axis_name="c", subcore_axis_name="s")

    @functools.partial(
        pl.kernel,
        mesh=mesh,
        out_type=jax.ShapeDtypeStruct((8, nfull, 8, 128), jnp.float32),
        scratch_types=[
            pltpu.VMEM((2, KT, 8, 128), jnp.float32),
            pltpu.SemaphoreType.DMA((2,)),
            pltpu.SemaphoreType.DMA((2,)),
        ],
        compiler_params=pltpu.CompilerParams(use_tc_tiling_on_sc=True,
                                             needs_layout_passes=False),
    )
    def k(tab_hbm, out_hbm, buf_v, gsem, osem):
        wid = lax.axis_index("s") * NC + lax.axis_index("c")
        nunits = 8 * nchunk

        def fire(u, k_):
            g = lax.rem(u, 8)
            i = lax.div(u, 8)
            for j in range(KT):
                pltpu.async_copy(
                    tab_hbm.at[pl.ds(g * 8, 8),
                               pl.ds((i * KT + j) * 128, 128)],
                    buf_v.at[k_, j], gsem.at[k_])

        fire(wid, 0)

        niters = nunits // NW

        @pl.loop(0, niters)
        def outer(it):
            u = it * NW + wid
            for kk in range(2):

                @pl.when(lax.rem(it, 2) == kk)
                def _work():
                    pltpu.make_async_copy(
                        out_hbm.at[0, pl.ds(0, KT), :, :],
                        buf_v.at[kk], gsem.at[kk]).wait()

                    @pl.when(it >= 1)
                    def _drain():
                        pltpu.make_async_copy(
                            buf_v.at[1 - kk],
                            out_hbm.at[0, pl.ds(0, KT), :, :],
                            osem.at[1 - kk]).wait()

                    @pl.when(it + 1 < niters)
                    def _pref():
                        fire(u + NW, 1 - kk)

                    g = lax.rem(u, 8)
                    i = lax.div(u, 8)
                    pltpu.async_copy(
                        buf_v.at[kk],
                        out_hbm.at[g, pl.ds(i * KT, KT), :, :],
                        osem.at[kk])

        pltpu.make_async_copy(
            buf_v.at[(niters - 1) % 2],
            out_hbm.at[0, pl.ds(0, KT), :, :],
            osem.at[(niters - 1) % 2]).wait()

        @pl.when(wid < 8 * rem)
        def _left():
            g = lax.rem(wid, 8)
            c = nchunk * KT + lax.div(wid, 8)
            pltpu.sync_copy(
                tab_hbm.at[pl.ds(g * 8, 8), pl.ds(c * 128, 128)],
                buf_v.at[0, 0])
            pltpu.sync_copy(buf_v.at[0, 0], out_hbm.at[g, c])

    return k(table_t)


def _transpose_blocks(x2, tail128):
    nfull = x2.shape[1]
    v = (nfull + 1) * 128
    mesh = plsc.VectorSubcoreMesh(core_axis_name="c", subcore_axis_name="s")

    @functools.partial(
        pl.kernel,
        mesh=mesh,
        out_type=jax.ShapeDtypeStruct((1000000 // 2, 128), jnp.float32),
        scratch_types=[
            pltpu.VMEM((2, 8, 8, TP), jnp.float32),
            pltpu.VMEM((2, 64, 128), jnp.float32),
            pltpu.VMEM((64, TP), jnp.float32),
            pltpu.SemaphoreType.DMA((2,)),
            pltpu.SemaphoreType.DMA((2,)),
        ],
        compiler_params=pltpu.CompilerParams(use_tc_tiling_on_sc=False,
                                             needs_layout_passes=False),
    )
    def k(x2_hbm, tail_hbm, out_hbm, slab_v, pack_v, tail_v, gsem, osem):
        wid = lax.axis_index("s") * NC + lax.axis_index("c")
        lane = jnp.arange(16, dtype=jnp.int32)
        g_hi = lane // 8
        d_lo = lane % 8

        def fire(blk, k_):
            pltpu.async_copy(
                x2_hbm.at[:, blk, :, :],
                slab_v.at[k_, :, :, pl.ds(0, 128)], gsem.at[k_])

        fire(wid, 0)

        niter = nfull // NW + 2

        @pl.loop(0, niter)
        def outer(i):
            blk = i * NW + wid
            for kk in range(2):

                @pl.when(jnp.logical_and(lax.rem(i, 2) == kk, blk < nfull))
                def _work():
                    nblk = blk + NW

                    @pl.when(nblk < nfull)
                    def _pref():
                        fire(nblk, 1 - kk)

                    pltpu.make_async_copy(
                        x2_hbm.at[:, 0, :, :],
                        slab_v.at[kk, :, :, pl.ds(0, 128)],
                        gsem.at[kk]).wait()

                    @pl.when(i >= 2)
                    def _drain():
                        pltpu.make_async_copy(
                            pack_v.at[kk], out_hbm.at[pl.ds(0, 64), :],
                            osem.at[kk]).wait()

                    @plsc.parallel_loop(0, 128, unroll=4)
                    def body(vv):
                        col = jnp.full((16,), vv, dtype=jnp.int32)
                        half = lax.rem(vv, 2) * 64
                        row = lax.div(vv, 2)
                        for c in range(4):
                            vals = plsc.load_gather(
                                slab_v.at[kk], [g_hi + 2 * c, d_lo, col])
                            pack_v[kk, row, pl.ds(half + c * 16, 16)] = vals

                    pltpu.async_copy(pack_v.at[kk],
                                     out_hbm.at[pl.ds(blk * 64, 64), :],
                                     osem.at[kk])

        for kk in range(2):
            pltpu.make_async_copy(
                pack_v.at[kk], out_hbm.at[pl.ds(0, 64), :],
                osem.at[kk]).wait()

        @pl.when(wid == 0)
        def _tail():
            pltpu.sync_copy(tail_hbm, tail_v.at[:, pl.ds(0, 128)])

            @plsc.parallel_loop(64, 128, unroll=4)
            def tail_body(vv):
                col = jnp.full((16,), vv, dtype=jnp.int32)
                half = lax.rem(vv, 2) * 64
                row = lax.div(vv, 2) - 32
                for c in range(4):
                    vals = plsc.load_gather(tail_v, [c * 16 + lane, col])
                    pack_v[0, row, pl.ds(half + c * 16, 16)] = vals

            pltpu.sync_copy(pack_v.at[0, pl.ds(0, 32), :],
                            out_hbm.at[pl.ds(nfull * 64, 32), :])

    return k(x2, tail128)


def _gather_scale(table, idx2d):
    ngroups = idx2d.shape[0]
    g_per_w = ngroups // NW
    sdim = 200
    btiles = 32
    dtiles = DIM // 8
    mesh = plsc.VectorSubcoreMesh(core_axis_name="c", subcore_axis_name="s")

    @functools.partial(
        pl.kernel,
        mesh=mesh,
        out_type=jax.ShapeDtypeStruct((sdim, dtiles, btiles, 8, GROUP),
                                      jnp.float32),
        scratch_types=[
            pltpu.VMEM((g_per_w, GROUP), jnp.int32),
            pltpu.VMEM((NROW, GROUP, DIM), jnp.float32),
            pltpu.VMEM((2, DIM, TP), jnp.float32),
            pltpu.SemaphoreType.DMA((NROW,)),
            pltpu.SemaphoreType.DMA((2,)),
        ],
        compiler_params=pltpu.CompilerParams(use_tc_tiling_on_sc=False,
                                             needs_layout_passes=False),
    )
    def k(table_hbm, idx_hbm, out_hbm, idx_all, rows_v, trans_v, gsem, osem):
        wid = lax.axis_index("s") * NC + lax.axis_index("c")
        r0 = wid * g_per_w

        pltpu.sync_copy(idx_hbm.at[pl.ds(r0, g_per_w)], idx_all)

        for q in range(LOOKAHEAD):
            pltpu.async_copy(table_hbm.at[idx_all.at[q]], rows_v.at[q],
                             gsem.at[q])

        lane = jnp.arange(16, dtype=jnp.int32)

        @pl.loop(0, g_per_w // NROW)
        def outer(p):
            for q in range(NROW):
                g = p * NROW + q
                tb = q % 2

                pltpu.make_async_copy(
                    table_hbm.at[pl.ds(0, GROUP)], rows_v.at[q],
                    gsem.at[q]).wait()

                nq = (q + LOOKAHEAD) % NROW
                gn = g + LOOKAHEAD

                @pl.when(gn < g_per_w)
                def _prefetch():
                    pltpu.async_copy(
                        table_hbm.at[idx_all.at[gn]], rows_v.at[nq],
                        gsem.at[nq])

                r = r0 + g
                ss = lax.rem(r, 8)
                bt = lax.rem(lax.div(r, 8), btiles)
                s = lax.div(r, 8 * btiles) * 8 + ss

                @pl.when(g >= 2)
                def _drain():
                    pltpu.make_async_copy(
                        trans_v.at[tb, :, pl.ds(0, GROUP)],
                        out_hbm.at[0, :, 0], osem.at[tb]).wait()

                @plsc.parallel_loop(0, GROUP, unroll=4)
                def transpose(t):
                    col = jnp.full((16,), t, dtype=jnp.int32)
                    for c in range(DIM // 16):
                        v = rows_v[q, t, pl.ds(c * 16, 16)] * _SCALE
                        plsc.store_scatter(
                            trans_v.at[tb], [c * 16 + lane, col], v)

                for dt in range(dtiles):
                    pltpu.async_copy(
                        trans_v.at[tb, pl.ds(dt * 8, 8), pl.ds(0, GROUP)],
                        out_hbm.at[s, dt, bt], osem.at[tb])

        for tb in range(2):
            pltpu.make_async_copy(
                trans_v.at[tb, :, pl.ds(0, GROUP)],
                out_hbm.at[0, :, 0], osem.at[tb]).wait()

    return k(table, idx2d)


@jax.jit
def _embed_lookup(table_t, tail128, idx2d):
    x2 = _restack_tiles(table_t)
    packed = _transpose_blocks(x2, tail128)
    table_dense = packed.reshape(table_t.shape[1], DIM)
    return _gather_scale(table_dense, idx2d)


def kernel(input_ids, embedding):
    nb, ns = input_ids.shape
    nv = embedding.shape[0]
    idx2d = (input_ids.T.reshape(ns // 8, 8, nb // 128, 128)
             .transpose(0, 2, 1, 3)
             .reshape(ns // 8 * (nb // 128) * 8, 128)
             .astype(jnp.int32))
    tail128 = embedding[nv - 128:, :].T
    out5d = _embed_lookup(embedding.T, tail128, idx2d)
    return out5d.transpose(2, 4, 0, 1, 3).reshape(nb, ns, DIM)


# speedup vs baseline: 2.1444x; 1.0016x over previous
import functools
import math

import jax
import jax.numpy as jnp
from jax import lax
from jax.experimental import pallas as pl
from jax.experimental.pallas import tpu as pltpu
from jax.experimental.pallas import tpu_sc as plsc

DIM = 64
_SCALE = math.sqrt(DIM)

NC = 2
NS = 16
NW = NC * NS

GROUP = 128
NROW = 4
LOOKAHEAD = 2
TP = 129
KT = 16


def _restack_tiles(table_t):
    v = table_t.shape[1]
    nfull = v // 128
    nchunk = nfull // KT
    rem = nfull - nchunk * KT
    mesh = plsc.VectorSubcoreMesh(core_axis_name="c", subcore_axis_name="s")

    @functools.partial(
        pl.kernel,
        mesh=mesh,
        out_type=jax.ShapeDtypeStruct((8, nfull, 8, 128), jnp.float32),
        scratch_types=[
            pltpu.VMEM((2, KT, 8, 128), jnp.float32),
            pltpu.SemaphoreType.DMA((2,)),
            pltpu.SemaphoreType.DMA((2,)),
        ],
        compiler_params=pltpu.CompilerParams(use_tc_tiling_on_sc=True,
                                             needs_layout_passes=False),
    )
    def k(tab_hbm, out_hbm, buf_v, gsem, osem):
        wid = lax.axis_index("s") * NC + lax.axis_index("c")
        nunits = 8 * nchunk

        def fire(u, k_):
            g = lax.rem(u, 8)
            i = lax.div(u, 8)
            for j in range(KT):
                pltpu.async_copy(
                    tab_hbm.at[pl.ds(g * 8, 8),
                               pl.ds((i * KT + j) * 128, 128)],
                    buf_v.at[k_, j], gsem.at[k_])

        fire(wid, 0)

        niters = nunits // NW

        @pl.loop(0, niters)
        def outer(it):
            u = it * NW + wid
            for kk in range(2):

                @pl.when(lax.rem(it, 2) == kk)
                def _work():
                    pltpu.make_async_copy(
                        out_hbm.at[0, pl.ds(0, KT), :, :],
                        buf_v.at[kk], gsem.at[kk]).wait()

                    @pl.when(it >= 1)
                    def _drain():
                        pltpu.make_async_copy(
                            buf_v.at[1 - kk],
                            out_hbm.at[0, pl.ds(0, KT), :, :],
                            osem.at[1 - kk]).wait()

                    @pl.when(it + 1 < niters)
                    def _pref():
                        fire(u + NW, 1 - kk)

                    g = lax.rem(u, 8)
                    i = lax.div(u, 8)
                    pltpu.async_copy(
                        buf_v.at[kk],
                        out_hbm.at[g, pl.ds(i * KT, KT), :, :],
                        osem.at[kk])

        pltpu.make_async_copy(
            buf_v.at[(niters - 1) % 2],
            out_hbm.at[0, pl.ds(0, KT), :, :],
            osem.at[(niters - 1) % 2]).wait()

        @pl.when(wid < 8 * rem)
        def _left():
            g = lax.rem(wid, 8)
            c = nchunk * KT + lax.div(wid, 8)
            pltpu.sync_copy(
                tab_hbm.at[pl.ds(g * 8, 8), pl.ds(c * 128, 128)],
                buf_v.at[0, 0])
            pltpu.sync_copy(buf_v.at[0, 0], out_hbm.at[g, c])

    return k(table_t)


def _transpose_blocks(x2, tail128):
    nfull = x2.shape[1]
    v = (nfull + 1) * 128
    mesh = plsc.VectorSubcoreMesh(core_axis_name="c", subcore_axis_name="s")

    @functools.partial(
        pl.kernel,
        mesh=mesh,
        out_type=jax.ShapeDtypeStruct((1000000 // 2, 128), jnp.float32),
        scratch_types=[
            pltpu.VMEM((2, 8, 8, TP), jnp.float32),
            pltpu.VMEM((2, 64, 128), jnp.float32),
            pltpu.VMEM((64, TP), jnp.float32),
            pltpu.SemaphoreType.DMA((2,)),
            pltpu.SemaphoreType.DMA((2,)),
        ],
        compiler_params=pltpu.CompilerParams(use_tc_tiling_on_sc=False,
                                             needs_layout_passes=False),
    )
    def k(x2_hbm, tail_hbm, out_hbm, slab_v, pack_v, tail_v, gsem, osem):
        wid = lax.axis_index("s") * NC + lax.axis_index("c")
        lane = jnp.arange(16, dtype=jnp.int32)
        g_hi = lane // 8
        d_lo = lane % 8

        def fire(blk, k_):
            pltpu.async_copy(
                x2_hbm.at[:, blk, :, :],
                slab_v.at[k_, :, :, pl.ds(0, 128)], gsem.at[k_])

        fire(wid, 0)

        niter = nfull // NW + 2

        @pl.loop(0, niter)
        def outer(i):
            blk = i * NW + wid
            for kk in range(2):

                @pl.when(jnp.logical_and(lax.rem(i, 2) == kk, blk < nfull))
                def _work():
                    nblk = blk + NW

                    @pl.when(nblk < nfull)
                    def _pref():
                        fire(nblk, 1 - kk)

                    pltpu.make_async_copy(
                        x2_hbm.at[:, 0, :, :],
                        slab_v.at[kk, :, :, pl.ds(0, 128)],
                        gsem.at[kk]).wait()

                    @pl.when(i >= 2)
                    def _drain():
                        pltpu.make_async_copy(
                            pack_v.at[kk], out_hbm.at[pl.ds(0, 64), :],
                            osem.at[kk]).wait()

                    @plsc.parallel_loop(0, 128, unroll=8)
                    def body(vv):
                        col = jnp.full((16,), vv, dtype=jnp.int32)
                        half = lax.rem(vv, 2) * 64
                        row = lax.div(vv, 2)
                        for c in range(4):
                            vals = plsc.load_gather(
                                slab_v.at[kk], [g_hi + 2 * c, d_lo, col])
                            pack_v[kk, row, pl.ds(half + c * 16, 16)] = vals

                    pltpu.async_copy(pack_v.at[kk],
                                     out_hbm.at[pl.ds(blk * 64, 64), :],
                                     osem.at[kk])

        for kk in range(2):
            pltpu.make_async_copy(
                pack_v.at[kk], out_hbm.at[pl.ds(0, 64), :],
                osem.at[kk]).wait()

        @pl.when(wid == 0)
        def _tail():
            pltpu.sync_copy(tail_hbm, tail_v.at[:, pl.ds(0, 128)])

            @plsc.parallel_loop(64, 128, unroll=4)
            def tail_body(vv):
                col = jnp.full((16,), vv, dtype=jnp.int32)
                half = lax.rem(vv, 2) * 64
                row = lax.div(vv, 2) - 32
                for c in range(4):
                    vals = plsc.load_gather(tail_v, [c * 16 + lane, col])
                    pack_v[0, row, pl.ds(half + c * 16, 16)] = vals

            pltpu.sync_copy(pack_v.at[0, pl.ds(0, 32), :],
                            out_hbm.at[pl.ds(nfull * 64, 32), :])

    return k(x2, tail128)


def _gather_scale(table, idx2d):
    ngroups = idx2d.shape[0]
    g_per_w = ngroups // NW
    sdim = 200
    btiles = 32
    dtiles = DIM // 8
    mesh = plsc.VectorSubcoreMesh(core_axis_name="c", subcore_axis_name="s")

    @functools.partial(
        pl.kernel,
        mesh=mesh,
        out_type=jax.ShapeDtypeStruct((sdim, dtiles, btiles, 8, GROUP),
                                      jnp.float32),
        scratch_types=[
            pltpu.VMEM((g_per_w, GROUP), jnp.int32),
            pltpu.VMEM((NROW, GROUP, DIM), jnp.float32),
            pltpu.VMEM((2, DIM, TP), jnp.float32),
            pltpu.SemaphoreType.DMA((NROW,)),
            pltpu.SemaphoreType.DMA((2,)),
        ],
        compiler_params=pltpu.CompilerParams(use_tc_tiling_on_sc=False,
                                             needs_layout_passes=False),
    )
    def k(table_hbm, idx_hbm, out_hbm, idx_all, rows_v, trans_v, gsem, osem):
        wid = lax.axis_index("s") * NC + lax.axis_index("c")
        r0 = wid * g_per_w

        pltpu.sync_copy(idx_hbm.at[pl.ds(r0, g_per_w)], idx_all)

        for q in range(LOOKAHEAD):
            pltpu.async_copy(table_hbm.at[idx_all.at[q]], rows_v.at[q],
                             gsem.at[q])

        lane = jnp.arange(16, dtype=jnp.int32)

        @pl.loop(0, g_per_w // NROW)
        def outer(p):
            for q in range(NROW):
                g = p * NROW + q
                tb = q % 2

                pltpu.make_async_copy(
                    table_hbm.at[pl.ds(0, GROUP)], rows_v.at[q],
                    gsem.at[q]).wait()

                nq = (q + LOOKAHEAD) % NROW
                gn = g + LOOKAHEAD

                @pl.when(gn < g_per_w)
                def _prefetch():
                    pltpu.async_copy(
                        table_hbm.at[idx_all.at[gn]], rows_v.at[nq],
                        gsem.at[nq])

                r = r0 + g
                ss = lax.rem(r, 8)
                bt = lax.rem(lax.div(r, 8), btiles)
                s = lax.div(r, 8 * btiles) * 8 + ss

                @pl.when(g >= 2)
                def _drain():
                    pltpu.make_async_copy(
                        trans_v.at[tb, :, pl.ds(0, GROUP)],
                        out_hbm.at[0, :, 0], osem.at[tb]).wait()

                @plsc.parallel_loop(0, GROUP, unroll=8)
                def transpose(t):
                    col = jnp.full((16,), t, dtype=jnp.int32)
                    for c in range(DIM // 16):
                        v = rows_v[q, t, pl.ds(c * 16, 16)] * _SCALE
                        plsc.store_scatter(
                            trans_v.at[tb], [c * 16 + lane, col], v)

                for dt in range(dtiles):
                    pltpu.async_copy(
                        trans_v.at[tb, pl.ds(dt * 8, 8), pl.ds(0, GROUP)],
                        out_hbm.at[s, dt, bt], osem.at[tb])

        for tb in range(2):
            pltpu.make_async_copy(
                trans_v.at[tb, :, pl.ds(0, GROUP)],
                out_hbm.at[0, :, 0], osem.at[tb]).wait()

    return k(table, idx2d)


@jax.jit
def _embed_lookup(table_t, tail128, idx2d):
    x2 = _restack_tiles(table_t)
    packed = _transpose_blocks(x2, tail128)
    table_dense = packed.reshape(table_t.shape[1], DIM)
    return _gather_scale(table_dense, idx2d)


def kernel(input_ids, embedding):
    nb, ns = input_ids.shape
    nv = embedding.shape[0]
    idx2d = (input_ids.T.reshape(ns // 8, 8, nb // 128, 128)
             .transpose(0, 2, 1, 3)
             .reshape(ns // 8 * (nb // 128) * 8, 128)
             .astype(jnp.int32))
    tail128 = embedding[nv - 128:, :].T
    out5d = _embed_lookup(embedding.T, tail128, idx2d)
    return out5d.transpose(2, 4, 0, 1, 3).reshape(nb, ns, DIM)


# speedup vs baseline: 2.2848x; 1.0655x over previous
import functools
import math

import jax
import jax.numpy as jnp
from jax import lax
from jax.experimental import pallas as pl
from jax.experimental.pallas import tpu as pltpu
from jax.experimental.pallas import tpu_sc as plsc

DIM = 64
_SCALE = math.sqrt(DIM)

NC = 2
NS = 16
NW = NC * NS

GROUP = 128
NROW = 4
LOOKAHEAD = 2
TP = 129
KT = 16


def _restack_tiles(table_t):
    v = table_t.shape[1]
    nfull = v // 128
    nchunk = nfull // KT
    rem = nfull - nchunk * KT
    mesh = plsc.VectorSubcoreMesh(core_axis_name="c", subcore_axis_name="s")

    @functools.partial(
        pl.kernel,
        mesh=mesh,
        out_type=jax.ShapeDtypeStruct((8, nfull, 8, 128), jnp.float32),
        scratch_types=[
            pltpu.VMEM((2, KT, 8, 128), jnp.float32),
            pltpu.SemaphoreType.DMA((2,)),
            pltpu.SemaphoreType.DMA((2,)),
        ],
        compiler_params=pltpu.CompilerParams(use_tc_tiling_on_sc=True,
                                             needs_layout_passes=False),
    )
    def k(tab_hbm, out_hbm, buf_v, gsem, osem):
        wid = lax.axis_index("s") * NC + lax.axis_index("c")
        nunits = 8 * nchunk

        def fire(u, k_):
            g = lax.rem(u, 8)
            i = lax.div(u, 8)
            for j in range(KT):
                pltpu.async_copy(
                    tab_hbm.at[pl.ds(g * 8, 8),
                               pl.ds((i * KT + j) * 128, 128)],
                    buf_v.at[k_, j], gsem.at[k_])

        fire(wid, 0)

        niters = nunits // NW

        @pl.loop(0, niters)
        def outer(it):
            u = it * NW + wid
            for kk in range(2):

                @pl.when(lax.rem(it, 2) == kk)
                def _work():
                    pltpu.make_async_copy(
                        out_hbm.at[0, pl.ds(0, KT), :, :],
                        buf_v.at[kk], gsem.at[kk]).wait()

                    @pl.when(it >= 1)
                    def _drain():
                        pltpu.make_async_copy(
                            buf_v.at[1 - kk],
                            out_hbm.at[0, pl.ds(0, KT), :, :],
                            osem.at[1 - kk]).wait()

                    @pl.when(it + 1 < niters)
                    def _pref():
                        fire(u + NW, 1 - kk)

                    g = lax.rem(u, 8)
                    i = lax.div(u, 8)
                    pltpu.async_copy(
                        buf_v.at[kk],
                        out_hbm.at[g, pl.ds(i * KT, KT), :, :],
                        osem.at[kk])

        pltpu.make_async_copy(
            buf_v.at[(niters - 1) % 2],
            out_hbm.at[0, pl.ds(0, KT), :, :],
            osem.at[(niters - 1) % 2]).wait()

        @pl.when(wid < 8 * rem)
        def _left():
            g = lax.rem(wid, 8)
            c = nchunk * KT + lax.div(wid, 8)
            pltpu.sync_copy(
                tab_hbm.at[pl.ds(g * 8, 8), pl.ds(c * 128, 128)],
                buf_v.at[0, 0])
            pltpu.sync_copy(buf_v.at[0, 0], out_hbm.at[g, c])

    return k(table_t)


def _transpose_blocks(x2, tail128):
    nfull = x2.shape[1]
    v = (nfull + 1) * 128
    mesh = plsc.VectorSubcoreMesh(core_axis_name="c", subcore_axis_name="s")

    @functools.partial(
        pl.kernel,
        mesh=mesh,
        out_type=jax.ShapeDtypeStruct((1000000 // 2, 128), jnp.float32),
        scratch_types=[
            pltpu.VMEM((3, 8, 8, TP), jnp.float32),
            pltpu.VMEM((3, 64, 128), jnp.float32),
            pltpu.VMEM((64, TP), jnp.float32),
            pltpu.SemaphoreType.DMA((3,)),
            pltpu.SemaphoreType.DMA((3,)),
        ],
        compiler_params=pltpu.CompilerParams(use_tc_tiling_on_sc=False,
                                             needs_layout_passes=False),
    )
    def k(x2_hbm, tail_hbm, out_hbm, slab_v, pack_v, tail_v, gsem, osem):
        wid = lax.axis_index("s") * NC + lax.axis_index("c")
        lane = jnp.arange(16, dtype=jnp.int32)
        g_hi = lane // 8
        d_lo = lane % 8

        def fire(blk, k_):
            pltpu.async_copy(
                x2_hbm.at[:, blk, :, :],
                slab_v.at[k_, :, :, pl.ds(0, 128)], gsem.at[k_])

        fire(wid, 0)
        fire(wid + NW, 1)

        niter = nfull // NW + 2

        @pl.loop(0, niter)
        def outer(i):
            blk = i * NW + wid
            for kk in range(3):

                @pl.when(jnp.logical_and(lax.rem(i, 3) == kk, blk < nfull))
                def _work():
                    nblk = blk + 2 * NW

                    @pl.when(nblk < nfull)
                    def _pref():
                        fire(nblk, (kk + 2) % 3)

                    pltpu.make_async_copy(
                        x2_hbm.at[:, 0, :, :],
                        slab_v.at[kk, :, :, pl.ds(0, 128)],
                        gsem.at[kk]).wait()

                    @pl.when(i >= 3)
                    def _drain():
                        pltpu.make_async_copy(
                            pack_v.at[kk], out_hbm.at[pl.ds(0, 64), :],
                            osem.at[kk]).wait()

                    @plsc.parallel_loop(0, 128, unroll=8)
                    def body(vv):
                        col = jnp.full((16,), vv, dtype=jnp.int32)
                        half = lax.rem(vv, 2) * 64
                        row = lax.div(vv, 2)
                        for c in range(4):
                            vals = plsc.load_gather(
                                slab_v.at[kk], [g_hi + 2 * c, d_lo, col])
                            pack_v[kk, row, pl.ds(half + c * 16, 16)] = vals

                    pltpu.async_copy(pack_v.at[kk],
                                     out_hbm.at[pl.ds(blk * 64, 64), :],
                                     osem.at[kk])

        for kk in range(3):
            pltpu.make_async_copy(
                pack_v.at[kk], out_hbm.at[pl.ds(0, 64), :],
                osem.at[kk]).wait()

        @pl.when(wid == 0)
        def _tail():
            pltpu.sync_copy(tail_hbm, tail_v.at[:, pl.ds(0, 128)])

            @plsc.parallel_loop(64, 128, unroll=4)
            def tail_body(vv):
                col = jnp.full((16,), vv, dtype=jnp.int32)
                half = lax.rem(vv, 2) * 64
                row = lax.div(vv, 2) - 32
                for c in range(4):
                    vals = plsc.load_gather(tail_v, [c * 16 + lane, col])
                    pack_v[0, row, pl.ds(half + c * 16, 16)] = vals

            pltpu.sync_copy(pack_v.at[0, pl.ds(0, 32), :],
                            out_hbm.at[pl.ds(nfull * 64, 32), :])

    return k(x2, tail128)


def _gather_scale(table, idx2d):
    ngroups = idx2d.shape[0]
    g_per_w = ngroups // NW
    sdim = 200
    btiles = 32
    dtiles = DIM // 8
    mesh = plsc.VectorSubcoreMesh(core_axis_name="c", subcore_axis_name="s")

    @functools.partial(
        pl.kernel,
        mesh=mesh,
        out_type=jax.ShapeDtypeStruct((sdim, dtiles, btiles, 8, GROUP),
                                      jnp.float32),
        scratch_types=[
            pltpu.VMEM((g_per_w, GROUP), jnp.int32),
            pltpu.VMEM((NROW, GROUP, DIM), jnp.float32),
            pltpu.VMEM((2, DIM, TP), jnp.float32),
            pltpu.SemaphoreType.DMA((NROW,)),
            pltpu.SemaphoreType.DMA((2,)),
        ],
        compiler_params=pltpu.CompilerParams(use_tc_tiling_on_sc=False,
                                             needs_layout_passes=False),
    )
    def k(table_hbm, idx_hbm, out_hbm, idx_all, rows_v, trans_v, gsem, osem):
        wid = lax.axis_index("s") * NC + lax.axis_index("c")
        r0 = wid * g_per_w

        pltpu.sync_copy(idx_hbm.at[pl.ds(r0, g_per_w)], idx_all)

        for q in range(LOOKAHEAD):
            pltpu.async_copy(table_hbm.at[idx_all.at[q]], rows_v.at[q],
                             gsem.at[q])

        lane = jnp.arange(16, dtype=jnp.int32)

        @pl.loop(0, g_per_w // NROW)
        def outer(p):
            for q in range(NROW):
                g = p * NROW + q
                tb = q % 2

                pltpu.make_async_copy(
                    table_hbm.at[pl.ds(0, GROUP)], rows_v.at[q],
                    gsem.at[q]).wait()

                nq = (q + LOOKAHEAD) % NROW
                gn = g + LOOKAHEAD

                @pl.when(gn < g_per_w)
                def _prefetch():
                    pltpu.async_copy(
                        table_hbm.at[idx_all.at[gn]], rows_v.at[nq],
                        gsem.at[nq])

                r = r0 + g
                ss = lax.rem(r, 8)
                bt = lax.rem(lax.div(r, 8), btiles)
                s = lax.div(r, 8 * btiles) * 8 + ss

                @pl.when(g >= 2)
                def _drain():
                    pltpu.make_async_copy(
                        trans_v.at[tb, :, pl.ds(0, GROUP)],
                        out_hbm.at[0, :, 0], osem.at[tb]).wait()

                @plsc.parallel_loop(0, GROUP, unroll=8)
                def transpose(t):
                    col = jnp.full((16,), t, dtype=jnp.int32)
                    for c in range(DIM // 16):
                        v = rows_v[q, t, pl.ds(c * 16, 16)] * _SCALE
                        plsc.store_scatter(
                            trans_v.at[tb], [c * 16 + lane, col], v)

                for dt in range(dtiles):
                    pltpu.async_copy(
                        trans_v.at[tb, pl.ds(dt * 8, 8), pl.ds(0, GROUP)],
                        out_hbm.at[s, dt, bt], osem.at[tb])

        for tb in range(2):
            pltpu.make_async_copy(
                trans_v.at[tb, :, pl.ds(0, GROUP)],
                out_hbm.at[0, :, 0], osem.at[tb]).wait()

    return k(table, idx2d)


@jax.jit
def _embed_lookup(table_t, tail128, idx2d):
    x2 = _restack_tiles(table_t)
    packed = _transpose_blocks(x2, tail128)
    table_dense = packed.reshape(table_t.shape[1], DIM)
    return _gather_scale(table_dense, idx2d)


def kernel(input_ids, embedding):
    nb, ns = input_ids.shape
    nv = embedding.shape[0]
    idx2d = (input_ids.T.reshape(ns // 8, 8, nb // 128, 128)
             .transpose(0, 2, 1, 3)
             .reshape(ns // 8 * (nb // 128) * 8, 128)
             .astype(jnp.int32))
    tail128 = embedding[nv - 128:, :].T
    out5d = _embed_lookup(embedding.T, tail128, idx2d)
    return out5d.transpose(2, 4, 0, 1, 3).reshape(nb, ns, DIM)
